# baseline (speedup 1.0000x reference)
import functools

import jax
import jax.numpy as jnp
from jax import lax
from jax.experimental import pallas as pl
from jax.experimental.pallas import tpu as pltpu
from jax.experimental.pallas import tpu_sc as plsc

VOCAB = 1000000
EMBED_DIM = 32
BATCH = 16384
HIST = 50

B = BATCH * HIST
NW = 32
BPW = B // NW
IPG = 512
G = BPW // IPG
K = 2
CK = K * IPG
NSC = G // K

_mesh = plsc.VectorSubcoreMesh(core_axis_name="c", subcore_axis_name="s")


@functools.partial(
    pl.kernel,
    mesh=_mesh,
    out_type=jax.ShapeDtypeStruct((B, EMBED_DIM), jnp.float32),
    scratch_types=[
        pltpu.VMEM((G, IPG), jnp.int32),
        pltpu.VMEM((CK, EMBED_DIM), jnp.float32),
        pltpu.SemaphoreType.DMA,
    ],
    compiler_params=pltpu.CompilerParams(use_tc_tiling_on_sc=False),
)
def _gather_kernel(table_hbm, idx_hbm, out_hbm, idx_v, rows_v, sem):
  wid = lax.axis_index("s") * 2 + lax.axis_index("c")
  base = wid * BPW
  pltpu.sync_copy(idx_hbm.at[wid], idx_v)

  def super_chunk(g, _):
    copies = []
    for j in range(K):
      copies.append(
          pltpu.async_copy(
              table_hbm.at[idx_v.at[g * K + j]],
              rows_v.at[pl.ds(j * IPG, IPG)],
              sem,
          )
      )
    for cp in copies:
      cp.wait()
    pltpu.sync_copy(rows_v, out_hbm.at[pl.ds(base + g * CK, CK)])
    return 0

  lax.fori_loop(0, NSC, super_chunk, 0)


def kernel(inputs, embeddings):
  idx = inputs.astype(jnp.int32).reshape(NW, G, IPG)
  out = _gather_kernel(embeddings, idx)
  return out.reshape(BATCH, HIST, EMBED_DIM)


# speedup vs baseline: 1.0063x; 1.0063x over previous
import functools

import jax
import jax.numpy as jnp
from jax import lax
from jax.experimental import pallas as pl
from jax.experimental.pallas import tpu as pltpu
from jax.experimental.pallas import tpu_sc as plsc

VOCAB = 1000000
EMBED_DIM = 32
BATCH = 16384
HIST = 50

B = BATCH * HIST
NW = 32
BPW = B // NW
IPG = 128
G = BPW // IPG
K = 10
CK = K * IPG
NSC = G // K
NPAIR = NSC // 2

_mesh = plsc.VectorSubcoreMesh(core_axis_name="c", subcore_axis_name="s")


@functools.partial(
    pl.kernel,
    mesh=_mesh,
    out_type=jax.ShapeDtypeStruct((B, EMBED_DIM), jnp.float32),
    scratch_types=[
        pltpu.VMEM((G, IPG), jnp.int32),
        pltpu.VMEM((2, CK, EMBED_DIM), jnp.float32),
        pltpu.SemaphoreType.DMA((2,)),
        pltpu.SemaphoreType.DMA((2,)),
    ],
    compiler_params=pltpu.CompilerParams(use_tc_tiling_on_sc=False),
)
def _gather_kernel(table_hbm, idx_hbm, out_hbm, idx_v, rows_v, gsem, wsem):
  wid = lax.axis_index("s") * 2 + lax.axis_index("c")
  base = wid * BPW
  pltpu.sync_copy(idx_hbm.at[wid], idx_v)

  def fire(g, buf):
    copies = []
    for j in range(K):
      copies.append(
          pltpu.async_copy(
              table_hbm.at[idx_v.at[g * K + j]],
              rows_v.at[buf].at[pl.ds(j * IPG, IPG)],
              gsem.at[buf],
          )
      )
    return copies

  fire(0, 0)
  fire(1, 1)

  def pair(h, _):
    g0 = 2 * h
    for buf in range(2):
      g = g0 + buf
      for j in range(K):
        pltpu.make_async_copy(
            table_hbm.at[idx_v.at[0]],
            rows_v.at[buf].at[pl.ds(j * IPG, IPG)],
            gsem.at[buf],
        ).wait()
      pltpu.async_copy(
          rows_v.at[buf],
          out_hbm.at[pl.ds(base + g * CK, CK)],
          wsem.at[buf],
      )
    for buf in range(2):
      g = g0 + buf
      pltpu.make_async_copy(
          rows_v.at[buf],
          out_hbm.at[pl.ds(base, CK)],
          wsem.at[buf],
      ).wait()

      @pl.when(g + 2 < NSC)
      def _():
        fire(g + 2, buf)

    return 0

  lax.fori_loop(0, NPAIR, pair, 0)


def kernel(inputs, embeddings):
  idx = inputs.astype(jnp.int32).reshape(NW, G, IPG)
  out = _gather_kernel(embeddings, idx)
  return out.reshape(BATCH, HIST, EMBED_DIM)
